# baseline (speedup 1.0000x reference)
import functools

import jax
import jax.numpy as jnp
from jax import lax
from jax.experimental import pallas as pl
from jax.experimental.pallas import tpu as pltpu
from jax.experimental.pallas import tpu_sc as plsc

N = 10000
E = 320000
D_FEAT = 128
D_EMB = 64
D_HID = 128
STEPS = 4
NUM_GRAPHS = 64

NC = 2
NS = 16
NW = NC * NS
CHUNK = 128
EDGES_PER_W = E // NW
NCHUNK = 80
HALF = NCHUNK // 2
EPW_PAD = NCHUNK * CHUNK
ROWS_PER_SUB = 632
NPAD = ROWS_PER_SUB * NS

ROW_BLK = 1000
GRID_N = N // ROW_BLK


def _input_body(x_ref, win_ref, wg_ref, h_ref, m_ref):
    h = jnp.tanh(jnp.dot(x_ref[...], win_ref[...],
                         preferred_element_type=jnp.float32))
    h_ref[...] = h
    m_ref[...] = jnp.dot(h, wg_ref[...], preferred_element_type=jnp.float32)


def _input_layer(x, win_pad, wg0):
    return pl.pallas_call(
        _input_body,
        grid=(GRID_N,),
        in_specs=[
            pl.BlockSpec((ROW_BLK, D_FEAT), lambda i: (i, 0)),
            pl.BlockSpec((D_FEAT, D_HID), lambda i: (0, 0)),
            pl.BlockSpec((D_HID, D_HID), lambda i: (0, 0)),
        ],
        out_specs=[
            pl.BlockSpec((ROW_BLK, D_HID), lambda i: (i, 0)),
            pl.BlockSpec((ROW_BLK, D_HID), lambda i: (i, 0)),
        ],
        out_shape=[
            jax.ShapeDtypeStruct((N, D_HID), jnp.float32),
            jax.ShapeDtypeStruct((N, D_HID), jnp.float32),
        ],
    )(x, win_pad, wg0)


def _gru_body(parts_ref, h_ref, wih_ref, whh_ref, bih_ref, bhh_ref, wg_ref,
              hn_ref, mn_ref):
    inp = parts_ref[0] + parts_ref[1]
    h = h_ref[...]
    gi = jnp.dot(inp, wih_ref[...], preferred_element_type=jnp.float32) + bih_ref[...]
    gh = jnp.dot(h, whh_ref[...], preferred_element_type=jnp.float32) + bhh_ref[...]
    i_r = gi[:, 0:D_HID]
    i_z = gi[:, D_HID:2 * D_HID]
    i_n = gi[:, 2 * D_HID:3 * D_HID]
    h_r = gh[:, 0:D_HID]
    h_z = gh[:, D_HID:2 * D_HID]
    h_n = gh[:, 2 * D_HID:3 * D_HID]
    r = jax.nn.sigmoid(i_r + h_r)
    z = jax.nn.sigmoid(i_z + h_z)
    n = jnp.tanh(i_n + r * h_n)
    hn = (1.0 - z) * n + z * h
    hn_ref[...] = hn
    mn_ref[...] = jnp.dot(hn, wg_ref[...], preferred_element_type=jnp.float32)


def _gru_step(parts, h, wih_t, whh_t, bih, bhh, wg_next):
    return pl.pallas_call(
        _gru_body,
        grid=(GRID_N,),
        in_specs=[
            pl.BlockSpec((NC, ROW_BLK, D_HID), lambda i: (0, i, 0)),
            pl.BlockSpec((ROW_BLK, D_HID), lambda i: (i, 0)),
            pl.BlockSpec((D_HID, 3 * D_HID), lambda i: (0, 0)),
            pl.BlockSpec((D_HID, 3 * D_HID), lambda i: (0, 0)),
            pl.BlockSpec((1, 3 * D_HID), lambda i: (0, 0)),
            pl.BlockSpec((1, 3 * D_HID), lambda i: (0, 0)),
            pl.BlockSpec((D_HID, D_HID), lambda i: (0, 0)),
        ],
        out_specs=[
            pl.BlockSpec((ROW_BLK, D_HID), lambda i: (i, 0)),
            pl.BlockSpec((ROW_BLK, D_HID), lambda i: (i, 0)),
        ],
        out_shape=[
            jax.ShapeDtypeStruct((N, D_HID), jnp.float32),
            jax.ShapeDtypeStruct((N, D_HID), jnp.float32),
        ],
    )(parts, h, wih_t, whh_t, bih, bhh, wg_next)


def _pool_body(batch_ref, h_ref, wpred_ref, bpred_ref, out_ref, acc, cnt):
    i = pl.program_id(0)

    @pl.when(i == 0)
    def _():
        acc[...] = jnp.zeros_like(acc)
        cnt[...] = jnp.zeros_like(cnt)

    b = batch_ref[0, 0, :]
    gids = lax.broadcasted_iota(jnp.int32, (ROW_BLK, NUM_GRAPHS), 1)
    p = (b[:, None] == gids).astype(jnp.float32)
    acc[...] += lax.dot_general(p, h_ref[...], (((0,), (0,)), ((), ())),
                                preferred_element_type=jnp.float32)
    cnt[...] += lax.dot_general(p, jnp.ones((ROW_BLK, D_HID), jnp.float32),
                                (((0,), (0,)), ((), ())),
                                preferred_element_type=jnp.float32)

    @pl.when(i == GRID_N - 1)
    def _():
        pooled = acc[...] / jnp.maximum(cnt[...], 1.0)
        pooled = jnp.maximum(pooled, 0.0)
        out_ref[...] = lax.dot_general(
            wpred_ref[...], pooled, (((1,), (1,)), ((), ())),
            preferred_element_type=jnp.float32) + bpred_ref[...]


def _pool(batch3, h, wpred, bpred2):
    return pl.pallas_call(
        _pool_body,
        grid=(GRID_N,),
        in_specs=[
            pl.BlockSpec((1, 1, ROW_BLK), lambda i: (i, 0, 0)),
            pl.BlockSpec((ROW_BLK, D_HID), lambda i: (i, 0)),
            pl.BlockSpec((1, D_HID), lambda i: (0, 0)),
            pl.BlockSpec((1, NUM_GRAPHS), lambda i: (0, 0)),
        ],
        out_specs=pl.BlockSpec((1, NUM_GRAPHS), lambda i: (0, 0)),
        out_shape=jax.ShapeDtypeStruct((1, NUM_GRAPHS), jnp.float32),
        scratch_shapes=[
            pltpu.VMEM((NUM_GRAPHS, D_HID), jnp.float32),
            pltpu.VMEM((NUM_GRAPHS, D_HID), jnp.float32),
        ],
    )(batch3, h, wpred, bpred2)


@functools.lru_cache(maxsize=None)
def _build_sc_spmm():
    mesh = plsc.VectorSubcoreMesh(core_axis_name="c", subcore_axis_name="s")

    @functools.partial(
        pl.kernel,
        out_type=jax.ShapeDtypeStruct((NC, NPAD, D_HID), jnp.float32),
        mesh=mesh,
        scratch_types=[
            pltpu.VMEM((CHUNK,), jnp.int32),
            pltpu.VMEM((CHUNK,), jnp.int32),
            pltpu.VMEM((CHUNK, D_HID), jnp.float32),
            pltpu.VMEM_SHARED((NPAD, D_HID), jnp.float32),
            pltpu.SemaphoreType.DMA,
        ],
    )
    def sc_spmm(m_hbm, src_hbm, dst_hbm, zeros_hbm, out_hbm,
                src_v, dst_v, rows_v, agg_sh, sem):
        c = lax.axis_index("c")
        s = lax.axis_index("s")
        row0 = s * ROWS_PER_SUB
        pltpu.sync_copy(zeros_hbm, agg_sh.at[pl.ds(row0, ROWS_PER_SUB)])
        plsc.subcore_barrier()

        def chunk(j, carry):
            pltpu.sync_copy(src_hbm.at[c, s, j], src_v)
            pltpu.sync_copy(dst_hbm.at[c, s, j], dst_v)
            pltpu.async_copy(m_hbm.at[src_v], rows_v, sem).wait()
            pltpu.sync_copy(rows_v, agg_sh.at[dst_v], add=True)
            return carry

        lax.fori_loop(0, NCHUNK, chunk, 0)
        plsc.subcore_barrier()
        pltpu.sync_copy(agg_sh.at[pl.ds(row0, ROWS_PER_SUB)],
                        out_hbm.at[c, pl.ds(row0, ROWS_PER_SUB)])

    return sc_spmm


def _sc_spmm(m, srcp, dstp, zeros):
    return _build_sc_spmm()(m, srcp, dstp, zeros)


def kernel(x, edge_index, batch, W_in, gg_weight, W_ih, W_hh, b_ih, b_hh,
           W_pred, b_pred):
    f32 = jnp.float32
    win_pad = jnp.pad(W_in.T.astype(f32), ((0, 0), (0, D_HID - D_EMB)))
    wih_t = W_ih.T.astype(f32)
    whh_t = W_hh.T.astype(f32)
    bih = b_ih.reshape(1, 3 * D_HID).astype(f32)
    bhh = b_hh.reshape(1, 3 * D_HID).astype(f32)
    bpred2 = jnp.broadcast_to(b_pred.reshape(1, 1),
                              (1, NUM_GRAPHS)).astype(f32)

    src = edge_index[0].reshape(NW, EDGES_PER_W)
    dst = edge_index[1].reshape(NW, EDGES_PER_W)
    pad = EPW_PAD - EDGES_PER_W
    srcp = jnp.pad(src, ((0, 0), (0, pad)),
                   constant_values=N).reshape(NC, NS, NCHUNK, CHUNK)
    pad_dst = jnp.broadcast_to(
        (jnp.arange(pad, dtype=jnp.int32) * 37) % N, (NW, pad))
    dstp = jnp.concatenate([dst, pad_dst], axis=1).reshape(
        NC, NS, NCHUNK, CHUNK)
    zeros = jnp.zeros((ROWS_PER_SUB, D_HID), f32)
    batch3 = batch.reshape(GRID_N, 1, ROW_BLK)

    h, m = _input_layer(x, win_pad, gg_weight[0])
    for i in range(STEPS):
        m_pad = jnp.pad(m, ((0, 8), (0, 0)))
        parts = _sc_spmm(m_pad, srcp, dstp, zeros)
        wg_next = gg_weight[i + 1] if i + 1 < STEPS else gg_weight[0]
        h, m = _gru_step(parts, h, wih_t, whh_t, bih, bhh, wg_next)
    out = _pool(batch3, h, W_pred, bpred2)
    return out[0, :]


# speedup vs baseline: 1.3947x; 1.3947x over previous
import functools

import jax
import jax.numpy as jnp
from jax import lax
from jax.experimental import pallas as pl
from jax.experimental.pallas import tpu as pltpu
from jax.experimental.pallas import tpu_sc as plsc

N = 10000
E = 320000
D_FEAT = 128
D_EMB = 64
D_HID = 128
STEPS = 4
NUM_GRAPHS = 64

NC = 2
NS = 16
NW = NC * NS
CHUNK = 128
EDGES_PER_W = E // NW
NCHUNK = 79
HALF = NCHUNK // 2
EPW_PAD = NCHUNK * CHUNK
ROWS_PER_SUB = 632
NPAD = ROWS_PER_SUB * NS

ROW_BLK = 1000
GRID_N = N // ROW_BLK


def _input_body(x_ref, win_ref, wg_ref, h_ref, m_ref):
    h = jnp.tanh(jnp.dot(x_ref[...], win_ref[...],
                         preferred_element_type=jnp.float32))
    h_ref[...] = h
    m_ref[...] = jnp.dot(h, wg_ref[...], preferred_element_type=jnp.float32)


def _input_layer(x, win_pad, wg0):
    return pl.pallas_call(
        _input_body,
        grid=(GRID_N,),
        in_specs=[
            pl.BlockSpec((ROW_BLK, D_FEAT), lambda i: (i, 0)),
            pl.BlockSpec((D_FEAT, D_HID), lambda i: (0, 0)),
            pl.BlockSpec((D_HID, D_HID), lambda i: (0, 0)),
        ],
        out_specs=[
            pl.BlockSpec((ROW_BLK, D_HID), lambda i: (i, 0)),
            pl.BlockSpec((ROW_BLK, D_HID), lambda i: (i, 0)),
        ],
        out_shape=[
            jax.ShapeDtypeStruct((N, D_HID), jnp.float32),
            jax.ShapeDtypeStruct((N, D_HID), jnp.float32),
        ],
    )(x, win_pad, wg0)


def _gru_body(parts_ref, h_ref, wih_ref, whh_ref, bih_ref, bhh_ref, wg_ref,
              hn_ref, mn_ref):
    inp = parts_ref[0] + parts_ref[1]
    h = h_ref[...]
    gi = jnp.dot(inp, wih_ref[...], preferred_element_type=jnp.float32) + bih_ref[...]
    gh = jnp.dot(h, whh_ref[...], preferred_element_type=jnp.float32) + bhh_ref[...]
    i_r = gi[:, 0:D_HID]
    i_z = gi[:, D_HID:2 * D_HID]
    i_n = gi[:, 2 * D_HID:3 * D_HID]
    h_r = gh[:, 0:D_HID]
    h_z = gh[:, D_HID:2 * D_HID]
    h_n = gh[:, 2 * D_HID:3 * D_HID]
    r = jax.nn.sigmoid(i_r + h_r)
    z = jax.nn.sigmoid(i_z + h_z)
    n = jnp.tanh(i_n + r * h_n)
    hn = (1.0 - z) * n + z * h
    hn_ref[...] = hn
    mn_ref[...] = jnp.dot(hn, wg_ref[...], preferred_element_type=jnp.float32)


def _gru_step(parts, h, wih_t, whh_t, bih, bhh, wg_next):
    return pl.pallas_call(
        _gru_body,
        grid=(GRID_N,),
        in_specs=[
            pl.BlockSpec((NC, ROW_BLK, D_HID), lambda i: (0, i, 0)),
            pl.BlockSpec((ROW_BLK, D_HID), lambda i: (i, 0)),
            pl.BlockSpec((D_HID, 3 * D_HID), lambda i: (0, 0)),
            pl.BlockSpec((D_HID, 3 * D_HID), lambda i: (0, 0)),
            pl.BlockSpec((1, 3 * D_HID), lambda i: (0, 0)),
            pl.BlockSpec((1, 3 * D_HID), lambda i: (0, 0)),
            pl.BlockSpec((D_HID, D_HID), lambda i: (0, 0)),
        ],
        out_specs=[
            pl.BlockSpec((ROW_BLK, D_HID), lambda i: (i, 0)),
            pl.BlockSpec((ROW_BLK, D_HID), lambda i: (i, 0)),
        ],
        out_shape=[
            jax.ShapeDtypeStruct((N, D_HID), jnp.float32),
            jax.ShapeDtypeStruct((N, D_HID), jnp.float32),
        ],
    )(parts, h, wih_t, whh_t, bih, bhh, wg_next)


def _pool_body(batch_ref, h_ref, wpred_ref, bpred_ref, out_ref, acc, cnt):
    i = pl.program_id(0)

    @pl.when(i == 0)
    def _():
        acc[...] = jnp.zeros_like(acc)
        cnt[...] = jnp.zeros_like(cnt)

    b = batch_ref[0, 0, :]
    gids = lax.broadcasted_iota(jnp.int32, (ROW_BLK, NUM_GRAPHS), 1)
    p = (b[:, None] == gids).astype(jnp.float32)
    acc[...] += lax.dot_general(p, h_ref[...], (((0,), (0,)), ((), ())),
                                preferred_element_type=jnp.float32)
    cnt[...] += lax.dot_general(p, jnp.ones((ROW_BLK, D_HID), jnp.float32),
                                (((0,), (0,)), ((), ())),
                                preferred_element_type=jnp.float32)

    @pl.when(i == GRID_N - 1)
    def _():
        pooled = acc[...] / jnp.maximum(cnt[...], 1.0)
        pooled = jnp.maximum(pooled, 0.0)
        out_ref[...] = lax.dot_general(
            wpred_ref[...], pooled, (((1,), (1,)), ((), ())),
            preferred_element_type=jnp.float32) + bpred_ref[...]


def _pool(batch3, h, wpred, bpred2):
    return pl.pallas_call(
        _pool_body,
        grid=(GRID_N,),
        in_specs=[
            pl.BlockSpec((1, 1, ROW_BLK), lambda i: (i, 0, 0)),
            pl.BlockSpec((ROW_BLK, D_HID), lambda i: (i, 0)),
            pl.BlockSpec((1, D_HID), lambda i: (0, 0)),
            pl.BlockSpec((1, NUM_GRAPHS), lambda i: (0, 0)),
        ],
        out_specs=pl.BlockSpec((1, NUM_GRAPHS), lambda i: (0, 0)),
        out_shape=jax.ShapeDtypeStruct((1, NUM_GRAPHS), jnp.float32),
        scratch_shapes=[
            pltpu.VMEM((NUM_GRAPHS, D_HID), jnp.float32),
            pltpu.VMEM((NUM_GRAPHS, D_HID), jnp.float32),
        ],
    )(batch3, h, wpred, bpred2)


@functools.lru_cache(maxsize=None)
def _build_sc_spmm():
    mesh = plsc.VectorSubcoreMesh(core_axis_name="c", subcore_axis_name="s")

    @functools.partial(
        pl.kernel,
        out_type=jax.ShapeDtypeStruct((NC, NPAD, D_HID), jnp.float32),
        mesh=mesh,
        scratch_types=[
            pltpu.VMEM((CHUNK,), jnp.int32),
            pltpu.VMEM((CHUNK,), jnp.int32),
            pltpu.VMEM((CHUNK, D_HID), jnp.float32),
            pltpu.VMEM_SHARED((NPAD, D_HID), jnp.float32),
            pltpu.SemaphoreType.DMA,
        ],
    )
    def sc_spmm(m_hbm, src_hbm, dst_hbm, zeros_hbm, out_hbm,
                src_v, dst_v, rows_v, agg_sh, sem):
        c = lax.axis_index("c")
        s = lax.axis_index("s")
        row0 = s * ROWS_PER_SUB
        pltpu.sync_copy(zeros_hbm, agg_sh.at[pl.ds(row0, ROWS_PER_SUB)])
        plsc.subcore_barrier()

        def chunk(j, carry):
            pltpu.sync_copy(src_hbm.at[c, s, j], src_v)
            pltpu.sync_copy(dst_hbm.at[c, s, j], dst_v)
            pltpu.async_copy(m_hbm.at[src_v], rows_v, sem).wait()
            pltpu.sync_copy(rows_v, agg_sh.at[dst_v], add=True)
            return carry

        lax.fori_loop(0, NCHUNK, chunk, 0)
        plsc.subcore_barrier()
        pltpu.sync_copy(agg_sh.at[pl.ds(row0, ROWS_PER_SUB)],
                        out_hbm.at[c, pl.ds(row0, ROWS_PER_SUB)])

    return sc_spmm


def _sc_spmm(m, srcp, dstp, zeros):
    return _build_sc_spmm()(m, srcp, dstp, zeros)


def kernel(x, edge_index, batch, W_in, gg_weight, W_ih, W_hh, b_ih, b_hh,
           W_pred, b_pred):
    f32 = jnp.float32
    win_pad = jnp.pad(W_in.T.astype(f32), ((0, 0), (0, D_HID - D_EMB)))
    wih_t = W_ih.T.astype(f32)
    whh_t = W_hh.T.astype(f32)
    bih = b_ih.reshape(1, 3 * D_HID).astype(f32)
    bhh = b_hh.reshape(1, 3 * D_HID).astype(f32)
    bpred2 = jnp.broadcast_to(b_pred.reshape(1, 1),
                              (1, NUM_GRAPHS)).astype(f32)

    src = edge_index[0].reshape(NW, EDGES_PER_W)
    dst = edge_index[1].reshape(NW, EDGES_PER_W)
    pad = EPW_PAD - EDGES_PER_W
    srcp = jnp.pad(src, ((0, 0), (0, pad))).reshape(NC, NS, NCHUNK, CHUNK)
    dstp = jnp.pad(dst, ((0, 0), (0, pad)),
                   constant_values=N).reshape(NC, NS, NCHUNK, CHUNK)
    zeros = jnp.zeros((ROWS_PER_SUB, D_HID), f32)
    batch3 = batch.reshape(GRID_N, 1, ROW_BLK)

    h, m = _input_layer(x, win_pad, gg_weight[0])
    for i in range(STEPS):
        parts = _sc_spmm(m, srcp, dstp, zeros)
        wg_next = gg_weight[i + 1] if i + 1 < STEPS else gg_weight[0]
        h, m = _gru_step(parts, h, wih_t, whh_t, bih, bhh, wg_next)
    out = _pool(batch3, h, W_pred, bpred2)
    return out[0, :]


# speedup vs baseline: 1.7925x; 1.2852x over previous
import functools

import jax
import jax.numpy as jnp
from jax import lax
from jax.experimental import pallas as pl
from jax.experimental.pallas import tpu as pltpu
from jax.experimental.pallas import tpu_sc as plsc

N = 10000
E = 320000
D_FEAT = 128
D_EMB = 64
D_HID = 128
STEPS = 4
NUM_GRAPHS = 64

NC = 2
NS = 16
NW = NC * NS
CHUNK = 128
EDGES_PER_W = E // NW
NCHUNK = 79
HALF = NCHUNK // 2
EPW_PAD = NCHUNK * CHUNK
ROWS_PER_SUB = 632
NPAD = ROWS_PER_SUB * NS

ROW_BLK = 1000
GRID_N = N // ROW_BLK


def _input_body(x_ref, win_ref, wg_ref, h_ref, m_ref):
    h = jnp.tanh(jnp.dot(x_ref[...], win_ref[...],
                         preferred_element_type=jnp.float32))
    h_ref[...] = h
    m_ref[...] = jnp.dot(h, wg_ref[...], preferred_element_type=jnp.float32)


def _input_layer(x, win_pad, wg0):
    return pl.pallas_call(
        _input_body,
        grid=(GRID_N,),
        in_specs=[
            pl.BlockSpec((ROW_BLK, D_FEAT), lambda i: (i, 0)),
            pl.BlockSpec((D_FEAT, D_HID), lambda i: (0, 0)),
            pl.BlockSpec((D_HID, D_HID), lambda i: (0, 0)),
        ],
        out_specs=[
            pl.BlockSpec((ROW_BLK, D_HID), lambda i: (i, 0)),
            pl.BlockSpec((ROW_BLK, D_HID), lambda i: (i, 0)),
        ],
        out_shape=[
            jax.ShapeDtypeStruct((N, D_HID), jnp.float32),
            jax.ShapeDtypeStruct((N, D_HID), jnp.float32),
        ],
    )(x, win_pad, wg0)


def _gru_body(parts_ref, h_ref, wih_ref, whh_ref, bih_ref, bhh_ref, wg_ref,
              hn_ref, mn_ref):
    inp = parts_ref[0] + parts_ref[1]
    h = h_ref[...]
    gi = jnp.dot(inp, wih_ref[...], preferred_element_type=jnp.float32) + bih_ref[...]
    gh = jnp.dot(h, whh_ref[...], preferred_element_type=jnp.float32) + bhh_ref[...]
    i_r = gi[:, 0:D_HID]
    i_z = gi[:, D_HID:2 * D_HID]
    i_n = gi[:, 2 * D_HID:3 * D_HID]
    h_r = gh[:, 0:D_HID]
    h_z = gh[:, D_HID:2 * D_HID]
    h_n = gh[:, 2 * D_HID:3 * D_HID]
    r = jax.nn.sigmoid(i_r + h_r)
    z = jax.nn.sigmoid(i_z + h_z)
    n = jnp.tanh(i_n + r * h_n)
    hn = (1.0 - z) * n + z * h
    hn_ref[...] = hn
    mn_ref[...] = jnp.dot(hn, wg_ref[...], preferred_element_type=jnp.float32)


def _gru_step(parts, h, wih_t, whh_t, bih, bhh, wg_next):
    return pl.pallas_call(
        _gru_body,
        grid=(GRID_N,),
        in_specs=[
            pl.BlockSpec((NC, ROW_BLK, D_HID), lambda i: (0, i, 0)),
            pl.BlockSpec((ROW_BLK, D_HID), lambda i: (i, 0)),
            pl.BlockSpec((D_HID, 3 * D_HID), lambda i: (0, 0)),
            pl.BlockSpec((D_HID, 3 * D_HID), lambda i: (0, 0)),
            pl.BlockSpec((1, 3 * D_HID), lambda i: (0, 0)),
            pl.BlockSpec((1, 3 * D_HID), lambda i: (0, 0)),
            pl.BlockSpec((D_HID, D_HID), lambda i: (0, 0)),
        ],
        out_specs=[
            pl.BlockSpec((ROW_BLK, D_HID), lambda i: (i, 0)),
            pl.BlockSpec((ROW_BLK, D_HID), lambda i: (i, 0)),
        ],
        out_shape=[
            jax.ShapeDtypeStruct((N, D_HID), jnp.float32),
            jax.ShapeDtypeStruct((N, D_HID), jnp.float32),
        ],
    )(parts, h, wih_t, whh_t, bih, bhh, wg_next)


def _pool_body(batch_ref, h_ref, wpred_ref, bpred_ref, out_ref, acc, cnt):
    i = pl.program_id(0)

    @pl.when(i == 0)
    def _():
        acc[...] = jnp.zeros_like(acc)
        cnt[...] = jnp.zeros_like(cnt)

    b = batch_ref[0, 0, :]
    gids = lax.broadcasted_iota(jnp.int32, (ROW_BLK, NUM_GRAPHS), 1)
    p = (b[:, None] == gids).astype(jnp.float32)
    acc[...] += lax.dot_general(p, h_ref[...], (((0,), (0,)), ((), ())),
                                preferred_element_type=jnp.float32)
    cnt[...] += lax.dot_general(p, jnp.ones((ROW_BLK, D_HID), jnp.float32),
                                (((0,), (0,)), ((), ())),
                                preferred_element_type=jnp.float32)

    @pl.when(i == GRID_N - 1)
    def _():
        pooled = acc[...] / jnp.maximum(cnt[...], 1.0)
        pooled = jnp.maximum(pooled, 0.0)
        out_ref[...] = lax.dot_general(
            wpred_ref[...], pooled, (((1,), (1,)), ((), ())),
            preferred_element_type=jnp.float32) + bpred_ref[...]


def _pool(batch3, h, wpred, bpred2):
    return pl.pallas_call(
        _pool_body,
        grid=(GRID_N,),
        in_specs=[
            pl.BlockSpec((1, 1, ROW_BLK), lambda i: (i, 0, 0)),
            pl.BlockSpec((ROW_BLK, D_HID), lambda i: (i, 0)),
            pl.BlockSpec((1, D_HID), lambda i: (0, 0)),
            pl.BlockSpec((1, NUM_GRAPHS), lambda i: (0, 0)),
        ],
        out_specs=pl.BlockSpec((1, NUM_GRAPHS), lambda i: (0, 0)),
        out_shape=jax.ShapeDtypeStruct((1, NUM_GRAPHS), jnp.float32),
        scratch_shapes=[
            pltpu.VMEM((NUM_GRAPHS, D_HID), jnp.float32),
            pltpu.VMEM((NUM_GRAPHS, D_HID), jnp.float32),
        ],
    )(batch3, h, wpred, bpred2)


@functools.lru_cache(maxsize=None)
def _build_sc_spmm():
    mesh = plsc.VectorSubcoreMesh(core_axis_name="c", subcore_axis_name="s")

    @functools.partial(
        pl.kernel,
        out_type=jax.ShapeDtypeStruct((NC, NPAD, D_HID), jnp.float32),
        mesh=mesh,
        scratch_types=[
            pltpu.VMEM((CHUNK,), jnp.int32),
            pltpu.VMEM((CHUNK,), jnp.int32),
            pltpu.VMEM((CHUNK,), jnp.int32),
            pltpu.VMEM((CHUNK,), jnp.int32),
            pltpu.VMEM((2, CHUNK, D_HID), jnp.float32),
            pltpu.VMEM_SHARED((NPAD, D_HID), jnp.float32),
            pltpu.SemaphoreType.DMA,
            pltpu.SemaphoreType.DMA,
        ],
    )
    def sc_spmm(m_hbm, src_hbm, dst_hbm, zeros_hbm, out_hbm,
                src0, src1, dst0, dst1, rows_v, agg_sh, gsem0, gsem1):
        c = lax.axis_index("c")
        s = lax.axis_index("s")
        row0 = s * ROWS_PER_SUB
        sbuf = (src0, src1)
        dbuf = (dst0, dst1)
        gsem = (gsem0, gsem1)

        def load_idx(j, b):
            pltpu.sync_copy(src_hbm.at[c, s, j], sbuf[b])
            pltpu.sync_copy(dst_hbm.at[c, s, j], dbuf[b])

        def gather(b):
            pltpu.async_copy(m_hbm.at[sbuf[b]], rows_v.at[b], gsem[b])

        def gather_wait(b):
            pltpu.make_async_copy(m_hbm.at[sbuf[b]], rows_v.at[b],
                                  gsem[b]).wait()

        def scatter(b):
            pltpu.sync_copy(rows_v.at[b], agg_sh.at[dbuf[b]], add=True)

        pltpu.sync_copy(zeros_hbm, agg_sh.at[pl.ds(row0, ROWS_PER_SUB)])
        plsc.subcore_barrier()

        load_idx(0, 0)
        gather(0)

        def pair(p, carry):
            j = 2 * p
            load_idx(j + 1, 1)
            gather(1)
            gather_wait(0)
            scatter(0)
            load_idx(j + 2, 0)
            gather(0)
            gather_wait(1)
            scatter(1)
            return carry

        lax.fori_loop(0, (NCHUNK - 1) // 2, pair, 0)
        gather_wait(0)
        scatter(0)
        plsc.subcore_barrier()
        pltpu.sync_copy(agg_sh.at[pl.ds(row0, ROWS_PER_SUB)],
                        out_hbm.at[c, pl.ds(row0, ROWS_PER_SUB)])

    return sc_spmm


def _sc_spmm(m, srcp, dstp, zeros):
    return _build_sc_spmm()(m, srcp, dstp, zeros)


def kernel(x, edge_index, batch, W_in, gg_weight, W_ih, W_hh, b_ih, b_hh,
           W_pred, b_pred):
    f32 = jnp.float32
    win_pad = jnp.pad(W_in.T.astype(f32), ((0, 0), (0, D_HID - D_EMB)))
    wih_t = W_ih.T.astype(f32)
    whh_t = W_hh.T.astype(f32)
    bih = b_ih.reshape(1, 3 * D_HID).astype(f32)
    bhh = b_hh.reshape(1, 3 * D_HID).astype(f32)
    bpred2 = jnp.broadcast_to(b_pred.reshape(1, 1),
                              (1, NUM_GRAPHS)).astype(f32)

    src = edge_index[0].reshape(NW, EDGES_PER_W)
    dst = edge_index[1].reshape(NW, EDGES_PER_W)
    pad = EPW_PAD - EDGES_PER_W
    srcp = jnp.pad(src, ((0, 0), (0, pad))).reshape(NC, NS, NCHUNK, CHUNK)
    dstp = jnp.pad(dst, ((0, 0), (0, pad)),
                   constant_values=N).reshape(NC, NS, NCHUNK, CHUNK)
    zeros = jnp.zeros((ROWS_PER_SUB, D_HID), f32)
    batch3 = batch.reshape(GRID_N, 1, ROW_BLK)

    h, m = _input_layer(x, win_pad, gg_weight[0])
    for i in range(STEPS):
        parts = _sc_spmm(m, srcp, dstp, zeros)
        wg_next = gg_weight[i + 1] if i + 1 < STEPS else gg_weight[0]
        h, m = _gru_step(parts, h, wih_t, whh_t, bih, bhh, wg_next)
    out = _pool(batch3, h, W_pred, bpred2)
    return out[0, :]


# speedup vs baseline: 1.9698x; 1.0989x over previous
import functools

import jax
import jax.numpy as jnp
from jax import lax
from jax.experimental import pallas as pl
from jax.experimental.pallas import tpu as pltpu
from jax.experimental.pallas import tpu_sc as plsc

N = 10000
E = 320000
D_FEAT = 128
D_EMB = 64
D_HID = 128
STEPS = 4
NUM_GRAPHS = 64

NC = 2
NS = 16
NW = NC * NS
CHUNK = 128
EDGES_PER_W = E // NW
NCHUNK = 79
HALF = NCHUNK // 2
EPW_PAD = NCHUNK * CHUNK
ROWS_PER_SUB = 632
NPAD = ROWS_PER_SUB * NS

ROW_BLK = 1000
GRID_N = N // ROW_BLK


def _input_body(x_ref, win_ref, wg_ref, h_ref, m_ref):
    h = jnp.tanh(jnp.dot(x_ref[...], win_ref[...],
                         preferred_element_type=jnp.float32))
    h_ref[...] = h
    m_ref[...] = jnp.dot(h, wg_ref[...], preferred_element_type=jnp.float32)


def _input_layer(x, win_pad, wg0):
    return pl.pallas_call(
        _input_body,
        grid=(GRID_N,),
        in_specs=[
            pl.BlockSpec((ROW_BLK, D_FEAT), lambda i: (i, 0)),
            pl.BlockSpec((D_FEAT, D_HID), lambda i: (0, 0)),
            pl.BlockSpec((D_HID, D_HID), lambda i: (0, 0)),
        ],
        out_specs=[
            pl.BlockSpec((ROW_BLK, D_HID), lambda i: (i, 0)),
            pl.BlockSpec((ROW_BLK, D_HID), lambda i: (i, 0)),
        ],
        out_shape=[
            jax.ShapeDtypeStruct((N, D_HID), jnp.float32),
            jax.ShapeDtypeStruct((N, D_HID), jnp.float32),
        ],
    )(x, win_pad, wg0)


def _gru_body(parts_ref, h_ref, wih_ref, whh_ref, bih_ref, bhh_ref, wg_ref,
              hn_ref, mn_ref):
    inp = parts_ref[0] + parts_ref[1]
    h = h_ref[...]
    gi = jnp.dot(inp, wih_ref[...], preferred_element_type=jnp.float32) + bih_ref[...]
    gh = jnp.dot(h, whh_ref[...], preferred_element_type=jnp.float32) + bhh_ref[...]
    i_r = gi[:, 0:D_HID]
    i_z = gi[:, D_HID:2 * D_HID]
    i_n = gi[:, 2 * D_HID:3 * D_HID]
    h_r = gh[:, 0:D_HID]
    h_z = gh[:, D_HID:2 * D_HID]
    h_n = gh[:, 2 * D_HID:3 * D_HID]
    r = jax.nn.sigmoid(i_r + h_r)
    z = jax.nn.sigmoid(i_z + h_z)
    n = jnp.tanh(i_n + r * h_n)
    hn = (1.0 - z) * n + z * h
    hn_ref[...] = hn
    mn_ref[...] = jnp.dot(hn, wg_ref[...], preferred_element_type=jnp.float32)


def _gru_step(parts, h, wih_t, whh_t, bih, bhh, wg_next):
    return pl.pallas_call(
        _gru_body,
        grid=(GRID_N,),
        in_specs=[
            pl.BlockSpec((NC, ROW_BLK, D_HID), lambda i: (0, i, 0)),
            pl.BlockSpec((ROW_BLK, D_HID), lambda i: (i, 0)),
            pl.BlockSpec((D_HID, 3 * D_HID), lambda i: (0, 0)),
            pl.BlockSpec((D_HID, 3 * D_HID), lambda i: (0, 0)),
            pl.BlockSpec((1, 3 * D_HID), lambda i: (0, 0)),
            pl.BlockSpec((1, 3 * D_HID), lambda i: (0, 0)),
            pl.BlockSpec((D_HID, D_HID), lambda i: (0, 0)),
        ],
        out_specs=[
            pl.BlockSpec((ROW_BLK, D_HID), lambda i: (i, 0)),
            pl.BlockSpec((ROW_BLK, D_HID), lambda i: (i, 0)),
        ],
        out_shape=[
            jax.ShapeDtypeStruct((N, D_HID), jnp.float32),
            jax.ShapeDtypeStruct((N, D_HID), jnp.float32),
        ],
    )(parts, h, wih_t, whh_t, bih, bhh, wg_next)


def _pool_body(batch_ref, h_ref, wpred_ref, bpred_ref, out_ref, acc, cnt):
    i = pl.program_id(0)

    @pl.when(i == 0)
    def _():
        acc[...] = jnp.zeros_like(acc)
        cnt[...] = jnp.zeros_like(cnt)

    b = batch_ref[0, 0, :]
    gids = lax.broadcasted_iota(jnp.int32, (ROW_BLK, NUM_GRAPHS), 1)
    p = (b[:, None] == gids).astype(jnp.float32)
    acc[...] += lax.dot_general(p, h_ref[...], (((0,), (0,)), ((), ())),
                                preferred_element_type=jnp.float32)
    cnt[...] += lax.dot_general(p, jnp.ones((ROW_BLK, D_HID), jnp.float32),
                                (((0,), (0,)), ((), ())),
                                preferred_element_type=jnp.float32)

    @pl.when(i == GRID_N - 1)
    def _():
        pooled = acc[...] / jnp.maximum(cnt[...], 1.0)
        pooled = jnp.maximum(pooled, 0.0)
        out_ref[...] = lax.dot_general(
            wpred_ref[...], pooled, (((1,), (1,)), ((), ())),
            preferred_element_type=jnp.float32) + bpred_ref[...]


def _pool(batch3, h, wpred, bpred2):
    return pl.pallas_call(
        _pool_body,
        grid=(GRID_N,),
        in_specs=[
            pl.BlockSpec((1, 1, ROW_BLK), lambda i: (i, 0, 0)),
            pl.BlockSpec((ROW_BLK, D_HID), lambda i: (i, 0)),
            pl.BlockSpec((1, D_HID), lambda i: (0, 0)),
            pl.BlockSpec((1, NUM_GRAPHS), lambda i: (0, 0)),
        ],
        out_specs=pl.BlockSpec((1, NUM_GRAPHS), lambda i: (0, 0)),
        out_shape=jax.ShapeDtypeStruct((1, NUM_GRAPHS), jnp.float32),
        scratch_shapes=[
            pltpu.VMEM((NUM_GRAPHS, D_HID), jnp.float32),
            pltpu.VMEM((NUM_GRAPHS, D_HID), jnp.float32),
        ],
    )(batch3, h, wpred, bpred2)


@functools.lru_cache(maxsize=None)
def _build_sc_spmm():
    mesh = plsc.VectorSubcoreMesh(core_axis_name="c", subcore_axis_name="s")

    @functools.partial(
        pl.kernel,
        out_type=jax.ShapeDtypeStruct((NC, NPAD, D_HID), jnp.float32),
        mesh=mesh,
        scratch_types=[
            pltpu.VMEM((1, CHUNK), jnp.int32),
            pltpu.VMEM((1, CHUNK), jnp.int32),
            pltpu.VMEM((1, CHUNK), jnp.int32),
            pltpu.VMEM((1, CHUNK), jnp.int32),
            pltpu.VMEM((2, CHUNK, D_HID), jnp.float32),
            pltpu.VMEM_SHARED((NPAD, D_HID), jnp.float32),
            pltpu.SemaphoreType.DMA,
            pltpu.SemaphoreType.DMA,
            pltpu.SemaphoreType.DMA,
            pltpu.SemaphoreType.DMA,
        ],
    )
    def sc_spmm(m_hbm, src_hbm, dst_hbm, zeros_hbm, out_hbm,
                src0, src1, dst0, dst1, rows_v, agg_sh,
                gsem0, gsem1, ssem0, ssem1):
        c = lax.axis_index("c")
        s = lax.axis_index("s")
        row0 = s * ROWS_PER_SUB
        sbuf = (src0, src1)
        dbuf = (dst0, dst1)
        gsem = (gsem0, gsem1)
        ssem = (ssem0, ssem1)

        def load_idx(j, b):
            pltpu.sync_copy(src_hbm.at[c, s, pl.ds(j, 1)], sbuf[b])
            pltpu.sync_copy(dst_hbm.at[c, s, pl.ds(j, 1)], dbuf[b])

        def gather(b):
            pltpu.async_copy(m_hbm.at[sbuf[b].at[0]], rows_v.at[b], gsem[b])

        def gather_wait(b):
            pltpu.make_async_copy(m_hbm.at[sbuf[b].at[0]], rows_v.at[b],
                                  gsem[b]).wait()

        def scatter(b):
            pltpu.async_copy(rows_v.at[b], agg_sh.at[dbuf[b].at[0]],
                             ssem[b], add=True)

        def scatter_wait(b):
            pltpu.make_async_copy(rows_v.at[b], agg_sh.at[dbuf[b].at[0]],
                                  ssem[b]).wait()

        def scatter_sync(b):
            pltpu.sync_copy(rows_v.at[b], agg_sh.at[dbuf[b].at[0]], add=True)

        pltpu.sync_copy(zeros_hbm, agg_sh.at[pl.ds(row0, ROWS_PER_SUB)])
        plsc.subcore_barrier()

        load_idx(0, 0)
        gather(0)
        load_idx(1, 1)
        gather(1)

        def pair(p, carry):
            j = 2 * p
            gather_wait(0)
            scatter(0)
            load_idx(j + 2, 0)
            scatter_wait(0)
            gather(0)
            gather_wait(1)
            scatter(1)
            load_idx(j + 3, 1)
            scatter_wait(1)
            gather(1)
            return carry

        lax.fori_loop(0, (NCHUNK - 3) // 2, pair, 0)
        gather_wait(0)
        scatter_sync(0)
        load_idx(NCHUNK - 1, 0)
        gather(0)
        gather_wait(1)
        scatter_sync(1)
        gather_wait(0)
        scatter_sync(0)
        plsc.subcore_barrier()
        pltpu.sync_copy(agg_sh.at[pl.ds(row0, ROWS_PER_SUB)],
                        out_hbm.at[c, pl.ds(row0, ROWS_PER_SUB)])

    return sc_spmm


def _sc_spmm(m, srcp, dstp, zeros):
    return _build_sc_spmm()(m, srcp, dstp, zeros)


def kernel(x, edge_index, batch, W_in, gg_weight, W_ih, W_hh, b_ih, b_hh,
           W_pred, b_pred):
    f32 = jnp.float32
    win_pad = jnp.pad(W_in.T.astype(f32), ((0, 0), (0, D_HID - D_EMB)))
    wih_t = W_ih.T.astype(f32)
    whh_t = W_hh.T.astype(f32)
    bih = b_ih.reshape(1, 3 * D_HID).astype(f32)
    bhh = b_hh.reshape(1, 3 * D_HID).astype(f32)
    bpred2 = jnp.broadcast_to(b_pred.reshape(1, 1),
                              (1, NUM_GRAPHS)).astype(f32)

    src = edge_index[0].reshape(NW, EDGES_PER_W)
    dst = edge_index[1].reshape(NW, EDGES_PER_W)
    pad = EPW_PAD - EDGES_PER_W
    srcp = jnp.pad(src, ((0, 0), (0, pad))).reshape(NC, NS, NCHUNK, CHUNK)
    dstp = jnp.pad(dst, ((0, 0), (0, pad)),
                   constant_values=N).reshape(NC, NS, NCHUNK, CHUNK)
    zeros = jnp.zeros((ROWS_PER_SUB, D_HID), f32)
    batch3 = batch.reshape(GRID_N, 1, ROW_BLK)

    h, m = _input_layer(x, win_pad, gg_weight[0])
    for i in range(STEPS):
        parts = _sc_spmm(m, srcp, dstp, zeros)
        wg_next = gg_weight[i + 1] if i + 1 < STEPS else gg_weight[0]
        h, m = _gru_step(parts, h, wih_t, whh_t, bih, bhh, wg_next)
    out = _pool(batch3, h, W_pred, bpred2)
    return out[0, :]


# speedup vs baseline: 2.0612x; 1.0464x over previous
import functools

import jax
import jax.numpy as jnp
from jax import lax
from jax.experimental import pallas as pl
from jax.experimental.pallas import tpu as pltpu
from jax.experimental.pallas import tpu_sc as plsc

N = 10000
E = 320000
D_FEAT = 128
D_EMB = 64
D_HID = 128
STEPS = 4
NUM_GRAPHS = 64

NC = 2
NS = 16
NW = NC * NS
CHUNK = 128
EDGES_PER_W = E // NW
NCHUNK = 79
HALF = NCHUNK // 2
EPW_PAD = NCHUNK * CHUNK
ROWS_PER_SUB = 632
NPAD = ROWS_PER_SUB * NS

ROW_BLK = 1000
GRID_N = N // ROW_BLK


def _input_body(x_ref, win_ref, wg_ref, h_ref, m_ref):
    h = jnp.tanh(jnp.dot(x_ref[...], win_ref[...],
                         preferred_element_type=jnp.float32))
    h_ref[...] = h
    m_ref[...] = jnp.dot(h, wg_ref[...], preferred_element_type=jnp.float32)


def _input_layer(x, win_pad, wg0):
    return pl.pallas_call(
        _input_body,
        grid=(GRID_N,),
        in_specs=[
            pl.BlockSpec((ROW_BLK, D_FEAT), lambda i: (i, 0)),
            pl.BlockSpec((D_FEAT, D_HID), lambda i: (0, 0)),
            pl.BlockSpec((D_HID, D_HID), lambda i: (0, 0)),
        ],
        out_specs=[
            pl.BlockSpec((ROW_BLK, D_HID), lambda i: (i, 0)),
            pl.BlockSpec((ROW_BLK, D_HID), lambda i: (i, 0)),
        ],
        out_shape=[
            jax.ShapeDtypeStruct((N, D_HID), jnp.float32),
            jax.ShapeDtypeStruct((N, D_HID), jnp.float32),
        ],
    )(x, win_pad, wg0)


def _gru_body(parts_ref, h_ref, wih_ref, whh_ref, bih_ref, bhh_ref, wg_ref,
              hn_ref, mn_ref):
    inp = parts_ref[0] + parts_ref[1]
    h = h_ref[...]
    gi = jnp.dot(inp, wih_ref[...], preferred_element_type=jnp.float32) + bih_ref[...]
    gh = jnp.dot(h, whh_ref[...], preferred_element_type=jnp.float32) + bhh_ref[...]
    i_r = gi[:, 0:D_HID]
    i_z = gi[:, D_HID:2 * D_HID]
    i_n = gi[:, 2 * D_HID:3 * D_HID]
    h_r = gh[:, 0:D_HID]
    h_z = gh[:, D_HID:2 * D_HID]
    h_n = gh[:, 2 * D_HID:3 * D_HID]
    r = jax.nn.sigmoid(i_r + h_r)
    z = jax.nn.sigmoid(i_z + h_z)
    n = jnp.tanh(i_n + r * h_n)
    hn = (1.0 - z) * n + z * h
    hn_ref[...] = hn
    mn_ref[...] = jnp.dot(hn, wg_ref[...], preferred_element_type=jnp.float32)


def _gru_step(parts, h, wih_t, whh_t, bih, bhh, wg_next):
    return pl.pallas_call(
        _gru_body,
        grid=(GRID_N,),
        in_specs=[
            pl.BlockSpec((NC, ROW_BLK, D_HID), lambda i: (0, i, 0)),
            pl.BlockSpec((ROW_BLK, D_HID), lambda i: (i, 0)),
            pl.BlockSpec((D_HID, 3 * D_HID), lambda i: (0, 0)),
            pl.BlockSpec((D_HID, 3 * D_HID), lambda i: (0, 0)),
            pl.BlockSpec((1, 3 * D_HID), lambda i: (0, 0)),
            pl.BlockSpec((1, 3 * D_HID), lambda i: (0, 0)),
            pl.BlockSpec((D_HID, D_HID), lambda i: (0, 0)),
        ],
        out_specs=[
            pl.BlockSpec((ROW_BLK, D_HID), lambda i: (i, 0)),
            pl.BlockSpec((ROW_BLK, D_HID), lambda i: (i, 0)),
        ],
        out_shape=[
            jax.ShapeDtypeStruct((N, D_HID), jnp.float32),
            jax.ShapeDtypeStruct((N, D_HID), jnp.float32),
        ],
    )(parts, h, wih_t, whh_t, bih, bhh, wg_next)


def _pool_body(batch_ref, h_ref, wpred_ref, bpred_ref, out_ref, acc, cnt):
    i = pl.program_id(0)

    @pl.when(i == 0)
    def _():
        acc[...] = jnp.zeros_like(acc)
        cnt[...] = jnp.zeros_like(cnt)

    b = batch_ref[0, 0, :]
    gids = lax.broadcasted_iota(jnp.int32, (ROW_BLK, NUM_GRAPHS), 1)
    p = (b[:, None] == gids).astype(jnp.float32)
    acc[...] += lax.dot_general(p, h_ref[...], (((0,), (0,)), ((), ())),
                                preferred_element_type=jnp.float32)
    cnt[...] += lax.dot_general(p, jnp.ones((ROW_BLK, D_HID), jnp.float32),
                                (((0,), (0,)), ((), ())),
                                preferred_element_type=jnp.float32)

    @pl.when(i == GRID_N - 1)
    def _():
        pooled = acc[...] / jnp.maximum(cnt[...], 1.0)
        pooled = jnp.maximum(pooled, 0.0)
        out_ref[...] = lax.dot_general(
            wpred_ref[...], pooled, (((1,), (1,)), ((), ())),
            preferred_element_type=jnp.float32) + bpred_ref[...]


def _pool(batch3, h, wpred, bpred2):
    return pl.pallas_call(
        _pool_body,
        grid=(GRID_N,),
        in_specs=[
            pl.BlockSpec((1, 1, ROW_BLK), lambda i: (i, 0, 0)),
            pl.BlockSpec((ROW_BLK, D_HID), lambda i: (i, 0)),
            pl.BlockSpec((1, D_HID), lambda i: (0, 0)),
            pl.BlockSpec((1, NUM_GRAPHS), lambda i: (0, 0)),
        ],
        out_specs=pl.BlockSpec((1, NUM_GRAPHS), lambda i: (0, 0)),
        out_shape=jax.ShapeDtypeStruct((1, NUM_GRAPHS), jnp.float32),
        scratch_shapes=[
            pltpu.VMEM((NUM_GRAPHS, D_HID), jnp.float32),
            pltpu.VMEM((NUM_GRAPHS, D_HID), jnp.float32),
        ],
    )(batch3, h, wpred, bpred2)


@functools.lru_cache(maxsize=None)
def _build_sc_spmm():
    mesh = plsc.VectorSubcoreMesh(core_axis_name="c", subcore_axis_name="s")

    @functools.partial(
        pl.kernel,
        out_type=jax.ShapeDtypeStruct((NC, NPAD, D_HID), jnp.float32),
        mesh=mesh,
        scratch_types=[
            pltpu.VMEM((2, CHUNK), jnp.int32),
            pltpu.VMEM((2, CHUNK), jnp.int32),
            pltpu.VMEM((2, CHUNK, D_HID), jnp.float32),
            pltpu.VMEM_SHARED((NPAD, D_HID), jnp.float32),
            pltpu.SemaphoreType.DMA,
            pltpu.SemaphoreType.DMA,
            pltpu.SemaphoreType.DMA,
            pltpu.SemaphoreType.DMA,
        ],
    )
    def sc_spmm(m_hbm, sd_hbm, zeros_hbm, out_hbm,
                sd0, sd1, rows_v, agg_sh,
                gsem0, gsem1, ssem0, ssem1):
        c = lax.axis_index("c")
        s = lax.axis_index("s")
        row0 = s * ROWS_PER_SUB
        sd = (sd0, sd1)
        gsem = (gsem0, gsem1)
        ssem = (ssem0, ssem1)

        def load_idx(j, b):
            pltpu.sync_copy(sd_hbm.at[c, s, j], sd[b])

        def gather(b):
            pltpu.async_copy(m_hbm.at[sd[b].at[0]], rows_v.at[b], gsem[b])

        def gather_wait(b):
            pltpu.make_async_copy(m_hbm.at[sd[b].at[0]], rows_v.at[b],
                                  gsem[b]).wait()

        def scatter(b):
            pltpu.async_copy(rows_v.at[b], agg_sh.at[sd[b].at[1]],
                             ssem[b], add=True)

        def scatter_wait(b):
            pltpu.make_async_copy(rows_v.at[b], agg_sh.at[sd[b].at[1]],
                                  ssem[b]).wait()

        def scatter_sync(b):
            pltpu.sync_copy(rows_v.at[b], agg_sh.at[sd[b].at[1]], add=True)

        pltpu.sync_copy(zeros_hbm, agg_sh.at[pl.ds(row0, ROWS_PER_SUB)])
        plsc.subcore_barrier()

        load_idx(0, 0)
        gather(0)
        load_idx(1, 1)
        gather(1)

        def pair(p, carry):
            j = 2 * p
            gather_wait(0)
            scatter(0)
            load_idx(j + 2, 0)
            scatter_wait(0)
            gather(0)
            gather_wait(1)
            scatter(1)
            load_idx(j + 3, 1)
            scatter_wait(1)
            gather(1)
            return carry

        lax.fori_loop(0, (NCHUNK - 3) // 2, pair, 0)
        gather_wait(0)
        scatter_sync(0)
        load_idx(NCHUNK - 1, 0)
        gather(0)
        gather_wait(1)
        scatter_sync(1)
        gather_wait(0)
        scatter_sync(0)
        plsc.subcore_barrier()
        pltpu.sync_copy(agg_sh.at[pl.ds(row0, ROWS_PER_SUB)],
                        out_hbm.at[c, pl.ds(row0, ROWS_PER_SUB)])

    return sc_spmm


def _sc_spmm(m, sdp, zeros):
    return _build_sc_spmm()(m, sdp, zeros)


def kernel(x, edge_index, batch, W_in, gg_weight, W_ih, W_hh, b_ih, b_hh,
           W_pred, b_pred):
    f32 = jnp.float32
    win_pad = jnp.pad(W_in.T.astype(f32), ((0, 0), (0, D_HID - D_EMB)))
    wih_t = W_ih.T.astype(f32)
    whh_t = W_hh.T.astype(f32)
    bih = b_ih.reshape(1, 3 * D_HID).astype(f32)
    bhh = b_hh.reshape(1, 3 * D_HID).astype(f32)
    bpred2 = jnp.broadcast_to(b_pred.reshape(1, 1),
                              (1, NUM_GRAPHS)).astype(f32)

    src = edge_index[0].reshape(NW, EDGES_PER_W)
    dst = edge_index[1].reshape(NW, EDGES_PER_W)
    pad = EPW_PAD - EDGES_PER_W
    srcp = jnp.pad(src, ((0, 0), (0, pad))).reshape(NC, NS, NCHUNK, 1, CHUNK)
    dstp = jnp.pad(dst, ((0, 0), (0, pad)),
                   constant_values=N).reshape(NC, NS, NCHUNK, 1, CHUNK)
    sdp = jnp.concatenate([srcp, dstp], axis=3)
    zeros = jnp.zeros((ROWS_PER_SUB, D_HID), f32)
    batch3 = batch.reshape(GRID_N, 1, ROW_BLK)

    h, m = _input_layer(x, win_pad, gg_weight[0])
    for i in range(STEPS):
        parts = _sc_spmm(m, sdp, zeros)
        wg_next = gg_weight[i + 1] if i + 1 < STEPS else gg_weight[0]
        h, m = _gru_step(parts, h, wih_t, whh_t, bih, bhh, wg_next)
    out = _pool(batch3, h, W_pred, bpred2)
    return out[0, :]
